# baseline (speedup 1.0000x reference)
import functools

import jax
import jax.numpy as jnp
from jax import lax
from jax.experimental import pallas as pl
from jax.experimental.pallas import tpu as pltpu
from jax.experimental.pallas import tpu_sc as plsc

B = 16384
SD = 16
ED = 64
IN_DIM = SD + 3 * ED
H1 = 512
H2 = 256
H3 = 128
NCLS = 5
EPS = 1e-5

NW = 32
BPW = B // NW
CHUNK = 128
NCHUNK = BPW // CHUNK

TB = 512
NT = B // TB


XW = 256
EMB = 3 * ED


def _sc_gather(pid2, rid2, aid2, E_p, E_r, E_a):
    mesh = plsc.VectorSubcoreMesh(core_axis_name="c", subcore_axis_name="s")

    @functools.partial(
        pl.kernel,
        mesh=mesh,
        out_type=jax.ShapeDtypeStruct((B, XW), jnp.bfloat16),
        scratch_types=[
            pltpu.VMEM((NCHUNK, CHUNK), jnp.int32),
            pltpu.VMEM((NCHUNK, CHUNK), jnp.int32),
            pltpu.VMEM((NCHUNK, CHUNK), jnp.int32),
            pltpu.VMEM((BPW, ED), jnp.bfloat16),
            pltpu.VMEM((BPW, ED), jnp.bfloat16),
            pltpu.VMEM((BPW, ED), jnp.bfloat16),
            pltpu.SemaphoreType.DMA,
        ],
        compiler_params=pltpu.CompilerParams(use_tc_tiling_on_sc=False),
    )
    def k(pid_h, rid_h, aid_h, ep_h, er_h, ea_h, x_h,
          pidx, ridx, aidx, prow, rrow, arow, sem):
        wid = lax.axis_index("s") * 2 + lax.axis_index("c")
        base = wid * BPW
        cbase = wid * NCHUNK
        pltpu.sync_copy(pid_h.at[pl.ds(cbase, NCHUNK)], pidx)
        pltpu.sync_copy(rid_h.at[pl.ds(cbase, NCHUNK)], ridx)
        pltpu.sync_copy(aid_h.at[pl.ds(cbase, NCHUNK)], aidx)
        copies = []
        for idx, tab, rows in ((pidx, ep_h, prow), (ridx, er_h, rrow),
                               (aidx, ea_h, arow)):
            for c in range(NCHUNK):
                copies.append(pltpu.async_copy(
                    tab.at[idx.at[c]], rows.at[pl.ds(c * CHUNK, CHUNK)], sem))
        for cp in copies:
            cp.wait()
        pltpu.sync_copy(prow, x_h.at[pl.ds(base, BPW), pl.ds(0, ED)])
        pltpu.sync_copy(rrow, x_h.at[pl.ds(base, BPW), pl.ds(ED, ED)])
        pltpu.sync_copy(arow, x_h.at[pl.ds(base, BPW), pl.ds(2 * ED, ED)])

    return k(pid2, rid2, aid2, E_p, E_r, E_a)


def _mlp_body(s_ref, x_ref, w1_ref, b1_ref, g1_ref, be1_ref,
              w2_ref, b2_ref, g2_ref, be2_ref, w3_ref, b3_ref, w4_ref, b4_ref,
              out_ref, s1, s2, h2s):
    p = pl.program_id(0)
    t = pl.program_id(1)
    inv_b = 1.0 / B

    def layer1():
        xb = jnp.concatenate(
            [s_ref[...].astype(jnp.bfloat16), x_ref[:, :EMB]], axis=1)
        h = jnp.dot(xb, w1_ref[...],
                    preferred_element_type=jnp.float32) + b1_ref[...]
        return jnp.maximum(h, 0.0)

    @pl.when(jnp.logical_and(p == 0, t == 0))
    def _():
        s1[...] = jnp.zeros_like(s1)

    @pl.when(p == 0)
    def _():
        h1 = layer1()
        s1[0:1, :] += jnp.sum(h1, axis=0, keepdims=True)
        s1[1:2, :] += jnp.sum(h1 * h1, axis=0, keepdims=True)

    @pl.when(jnp.logical_and(p == 1, t == 0))
    def _():
        s2[...] = jnp.zeros_like(s2)

    @pl.when(p == 1)
    def _():
        h1 = layer1()
        mean = s1[0:1, :] * inv_b
        var = s1[1:2, :] * inv_b - mean * mean
        scale = g1_ref[...] * lax.rsqrt(var + EPS)
        shift = be1_ref[...] - mean * scale
        h1 = h1 * scale + shift
        h2 = jnp.maximum(
            jnp.dot(h1.astype(jnp.bfloat16), w2_ref[...],
                    preferred_element_type=jnp.float32)
            + b2_ref[...], 0.0)
        h2s[pl.ds(t * TB, TB), :] = h2
        s2[0:1, :] += jnp.sum(h2, axis=0, keepdims=True)
        s2[1:2, :] += jnp.sum(h2 * h2, axis=0, keepdims=True)

    @pl.when(p == 2)
    def _():
        h2 = h2s[pl.ds(t * TB, TB), :]
        mean = s2[0:1, :] * inv_b
        var = s2[1:2, :] * inv_b - mean * mean
        scale = g2_ref[...] * lax.rsqrt(var + EPS)
        shift = be2_ref[...] - mean * scale
        h2 = h2 * scale + shift
        h3 = jnp.maximum(
            jnp.dot(h2.astype(jnp.bfloat16), w3_ref[...],
                    preferred_element_type=jnp.float32)
            + b3_ref[...], 0.0)
        out_ref[...] = (
            jnp.dot(h3.astype(jnp.bfloat16), w4_ref[...],
                    preferred_element_type=jnp.float32)
            + b4_ref[...])


def _mlp(sensor, x, W1, b1, g1, be1, W2, b2, g2, be2, W3, b3, W4, b4):
    def xmap(p, t):
        return (jnp.where(p < 2, t, NT - 1), 0)

    def const(p, t):
        return (0, 0)

    full = lambda arr: pl.BlockSpec(arr.shape, const)
    return pl.pallas_call(
        _mlp_body,
        grid=(3, NT),
        in_specs=[
            pl.BlockSpec((TB, SD), xmap),
            pl.BlockSpec((TB, XW), xmap),
            full(W1), full(b1), full(g1), full(be1),
            full(W2), full(b2), full(g2), full(be2),
            full(W3), full(b3), full(W4), full(b4),
        ],
        out_specs=pl.BlockSpec((TB, NCLS), lambda p, t: (t, 0)),
        out_shape=jax.ShapeDtypeStruct((B, NCLS), jnp.float32),
        scratch_shapes=[
            pltpu.VMEM((2, H1), jnp.float32),
            pltpu.VMEM((2, H2), jnp.float32),
            pltpu.VMEM((B, H2), jnp.float32),
        ],
    )(sensor, x, W1, b1, g1, be1, W2, b2, g2, be2, W3, b3, W4, b4)


def kernel(sensor_data, product_ids, recipe_ids, asset_ids, E_p, E_r, E_a,
           W1, b1, g1, be1, W2, b2, g2, be2, W3, b3, W4, b4):
    pid2 = product_ids.astype(jnp.int32).reshape(NW * NCHUNK, CHUNK)
    rid2 = recipe_ids.astype(jnp.int32).reshape(NW * NCHUNK, CHUNK)
    aid2 = asset_ids.astype(jnp.int32).reshape(NW * NCHUNK, CHUNK)
    x = _sc_gather(pid2, rid2, aid2, E_p.astype(jnp.bfloat16),
                   E_r.astype(jnp.bfloat16), E_a.astype(jnp.bfloat16))
    return _mlp(
        sensor_data, x, W1.astype(jnp.bfloat16),
        b1.reshape(1, H1), g1.reshape(1, H1), be1.reshape(1, H1),
        W2.astype(jnp.bfloat16), b2.reshape(1, H2), g2.reshape(1, H2),
        be2.reshape(1, H2), W3.astype(jnp.bfloat16), b3.reshape(1, H3),
        W4.astype(jnp.bfloat16), b4.reshape(1, NCLS))


# speedup vs baseline: 1.3306x; 1.3306x over previous
import functools

import jax
import jax.numpy as jnp
from jax import lax
from jax.experimental import pallas as pl
from jax.experimental.pallas import tpu as pltpu
from jax.experimental.pallas import tpu_sc as plsc

B = 16384
SD = 16
ED = 64
IN_DIM = SD + 3 * ED
H1 = 512
H2 = 256
H3 = 128
NCLS = 5
EPS = 1e-5

NW = 32
BPW = B // NW
CHUNK = 128
NCHUNK = BPW // CHUNK

TB = 1024
NT = B // TB

XW = 256


def _sc_gather(sensor_data, pid2, rid2, aid2, E_p, E_r, E_a):
    mesh = plsc.VectorSubcoreMesh(core_axis_name="c", subcore_axis_name="s")

    @functools.partial(
        pl.kernel,
        mesh=mesh,
        out_type=jax.ShapeDtypeStruct((B, XW), jnp.float32),
        scratch_types=[
            pltpu.VMEM((NCHUNK, CHUNK), jnp.int32),
            pltpu.VMEM((NCHUNK, CHUNK), jnp.int32),
            pltpu.VMEM((NCHUNK, CHUNK), jnp.int32),
            pltpu.VMEM((BPW, ED), jnp.float32),
            pltpu.VMEM((BPW, ED), jnp.float32),
            pltpu.VMEM((BPW, ED), jnp.float32),
            pltpu.VMEM((BPW, SD), jnp.float32),
            pltpu.SemaphoreType.DMA,
        ],
        compiler_params=pltpu.CompilerParams(use_tc_tiling_on_sc=False),
    )
    def k(sensor_h, pid_h, rid_h, aid_h, ep_h, er_h, ea_h, x_h,
          pidx, ridx, aidx, prow, rrow, arow, srow, sem):
        wid = lax.axis_index("s") * 2 + lax.axis_index("c")
        base = wid * BPW
        cbase = wid * NCHUNK
        pltpu.sync_copy(pid_h.at[pl.ds(cbase, NCHUNK)], pidx)
        pltpu.sync_copy(rid_h.at[pl.ds(cbase, NCHUNK)], ridx)
        pltpu.sync_copy(aid_h.at[pl.ds(cbase, NCHUNK)], aidx)
        pltpu.sync_copy(sensor_h.at[pl.ds(base, BPW)], srow)
        copies = []
        for idx, tab, rows in ((pidx, ep_h, prow), (ridx, er_h, rrow),
                               (aidx, ea_h, arow)):
            for c in range(NCHUNK):
                copies.append(pltpu.async_copy(
                    tab.at[idx.at[c]], rows.at[pl.ds(c * CHUNK, CHUNK)], sem))
        for cp in copies:
            cp.wait()
        pltpu.sync_copy(srow, x_h.at[pl.ds(base, BPW), pl.ds(0, SD)])
        pltpu.sync_copy(prow, x_h.at[pl.ds(base, BPW), pl.ds(SD, ED)])
        pltpu.sync_copy(rrow, x_h.at[pl.ds(base, BPW), pl.ds(SD + ED, ED)])
        pltpu.sync_copy(arow, x_h.at[pl.ds(base, BPW), pl.ds(SD + 2 * ED, ED)])

    return k(sensor_data, pid2, rid2, aid2, E_p, E_r, E_a)


def _mlp_body(x_hbm, w1_ref, b1_ref, g1_ref, be1_ref,
              w2_ref, b2_ref, g2_ref, be2_ref, w3_ref, b3_ref, w4_ref, b4_ref,
              out_ref, s1, s2, h2s, xbuf, sems):
    p = pl.program_id(0)
    t = pl.program_id(1)
    inv_b = 1.0 / B
    step = p * NT + t

    def issue(nstep):
        sl = lax.rem(nstep, 2)
        nt_ = lax.rem(nstep, NT)

        @pl.when(nstep < 2 * NT)
        def _():
            pltpu.make_async_copy(
                x_hbm.at[pl.ds(nt_ * TB, TB)], xbuf.at[sl],
                sems.at[sl]).start()

    @pl.when(step == 0)
    def _():
        issue(step)

    issue(step + 1)

    def layer1():
        sl = lax.rem(step, 2)
        pltpu.make_async_copy(
            x_hbm.at[pl.ds(t * TB, TB)], xbuf.at[sl], sems.at[sl]).wait()
        xv = xbuf[sl]
        h = jnp.dot(xv[:, :IN_DIM].astype(jnp.bfloat16), w1_ref[...],
                    preferred_element_type=jnp.float32) + b1_ref[...]
        return jnp.maximum(h, 0.0)

    @pl.when(jnp.logical_and(p == 0, t == 0))
    def _():
        s1[...] = jnp.zeros_like(s1)

    @pl.when(p == 0)
    def _():
        h1 = layer1()
        s1[0:1, :] += jnp.sum(h1, axis=0, keepdims=True)
        s1[1:2, :] += jnp.sum(h1 * h1, axis=0, keepdims=True)

    @pl.when(jnp.logical_and(p == 1, t == 0))
    def _():
        s2[...] = jnp.zeros_like(s2)

    @pl.when(p == 1)
    def _():
        h1 = layer1()
        mean = s1[0:1, :] * inv_b
        var = s1[1:2, :] * inv_b - mean * mean
        scale = g1_ref[...] * lax.rsqrt(var + EPS)
        shift = be1_ref[...] - mean * scale
        h1 = h1 * scale + shift
        h2 = jnp.maximum(
            jnp.dot(h1.astype(jnp.bfloat16), w2_ref[...],
                    preferred_element_type=jnp.float32)
            + b2_ref[...], 0.0)
        h2s[pl.ds(t * TB, TB), :] = h2
        s2[0:1, :] += jnp.sum(h2, axis=0, keepdims=True)
        s2[1:2, :] += jnp.sum(h2 * h2, axis=0, keepdims=True)

    @pl.when(p == 2)
    def _():
        h2 = h2s[pl.ds(t * TB, TB), :]
        mean = s2[0:1, :] * inv_b
        var = s2[1:2, :] * inv_b - mean * mean
        scale = g2_ref[...] * lax.rsqrt(var + EPS)
        shift = be2_ref[...] - mean * scale
        h2 = h2 * scale + shift
        h3 = jnp.maximum(
            jnp.dot(h2.astype(jnp.bfloat16), w3_ref[...],
                    preferred_element_type=jnp.float32)
            + b3_ref[...], 0.0)
        out_ref[...] = (
            jnp.dot(h3.astype(jnp.bfloat16), w4_ref[...],
                    preferred_element_type=jnp.float32)
            + b4_ref[...])


def _mlp(x, W1, b1, g1, be1, W2, b2, g2, be2, W3, b3, W4, b4):
    def const(p, t):
        return (0, 0)

    full = lambda arr: pl.BlockSpec(arr.shape, const)
    return pl.pallas_call(
        _mlp_body,
        grid=(3, NT),
        in_specs=[
            pl.BlockSpec(memory_space=pl.ANY),
            full(W1), full(b1), full(g1), full(be1),
            full(W2), full(b2), full(g2), full(be2),
            full(W3), full(b3), full(W4), full(b4),
        ],
        out_specs=pl.BlockSpec((TB, H3), lambda p, t: (t, 0)),
        out_shape=jax.ShapeDtypeStruct((B, H3), jnp.float32),
        scratch_shapes=[
            pltpu.VMEM((2, H1), jnp.float32),
            pltpu.VMEM((2, H2), jnp.float32),
            pltpu.VMEM((B, H2), jnp.float32),
            pltpu.VMEM((2, TB, XW), jnp.float32),
            pltpu.SemaphoreType.DMA((2,)),
        ],
    )(x, W1, b1, g1, be1, W2, b2, g2, be2, W3, b3, W4, b4)


def kernel(sensor_data, product_ids, recipe_ids, asset_ids, E_p, E_r, E_a,
           W1, b1, g1, be1, W2, b2, g2, be2, W3, b3, W4, b4):
    pid2 = product_ids.astype(jnp.int32).reshape(NW * NCHUNK, CHUNK)
    rid2 = recipe_ids.astype(jnp.int32).reshape(NW * NCHUNK, CHUNK)
    aid2 = asset_ids.astype(jnp.int32).reshape(NW * NCHUNK, CHUNK)
    x = _sc_gather(sensor_data, pid2, rid2, aid2, E_p, E_r, E_a)
    pad4 = jnp.zeros((H3 - NCLS,), dtype=jnp.float32)
    w4p = jnp.pad(W4, ((0, 0), (0, H3 - NCLS)))
    logits = _mlp(
        x, W1.astype(jnp.bfloat16),
        b1.reshape(1, H1), g1.reshape(1, H1), be1.reshape(1, H1),
        W2.astype(jnp.bfloat16), b2.reshape(1, H2), g2.reshape(1, H2),
        be2.reshape(1, H2), W3.astype(jnp.bfloat16), b3.reshape(1, H3),
        w4p.astype(jnp.bfloat16),
        jnp.concatenate([b4, pad4]).reshape(1, H3))
    return logits[:, :NCLS]
